# baseline (speedup 1.0000x reference)
import functools

import jax
import jax.numpy as jnp
from jax.experimental import pallas as pl
from jax.experimental.pallas import tpu as pltpu

_RATIO = 0.07651303708553314
_H_OFFSETS = (4, 2)
_W_COLS = (3, 1)

_CH = 16
_NBUF = 8


def _noise_points(h, w):
    mult = {}
    for ho, wv in zip(_H_OFFSETS, _W_COLS):
        f = (h - 5 + ho) * w + wv
        mult[f] = mult.get(f, 0) + 1
    return tuple(sorted(mult.items()))


def _body(points, nchunks, x_hbm, o_hbm, ibuf, obuf, lsem, ssem):
    def load(i):
        return pltpu.make_async_copy(
            x_hbm.at[pl.ds(i * _CH, _CH)], ibuf.at[i % _NBUF],
            lsem.at[i % _NBUF])

    def store(i):
        return pltpu.make_async_copy(
            obuf.at[i % _NBUF], o_hbm.at[pl.ds(i * _CH, _CH)],
            ssem.at[i % _NBUF])

    for j in range(min(_NBUF, nchunks)):
        load(j).start()

    for i in range(nchunks):
        s = i % _NBUF
        load(i).wait()
        if i >= _NBUF:
            store(i - _NBUF).wait()
        v = ibuf[s]
        m = jnp.max(v, axis=1, keepdims=True) * jnp.float32(_RATIO)
        obuf[s] = v
        for col, k in points:
            obuf[s, :, col:col + 1] = v[:, col:col + 1] + jnp.float32(k) * m
        store(i).start()
        if i + _NBUF < nchunks:
            load(i + _NBUF).start()

    for i in range(max(nchunks - _NBUF, 0), nchunks):
        store(i).wait()


@jax.jit
def kernel(x):
    b, c, h, w = x.shape
    n = b * c
    sp = h * w
    points = _noise_points(h, w)
    assert n % _CH == 0
    nchunks = n // _CH

    x2 = x.reshape(n, sp)
    y2 = pl.pallas_call(
        functools.partial(_body, points, nchunks),
        in_specs=[pl.BlockSpec(memory_space=pl.ANY)],
        out_specs=pl.BlockSpec(memory_space=pl.ANY),
        out_shape=jax.ShapeDtypeStruct((n, sp), x.dtype),
        scratch_shapes=[
            pltpu.VMEM((_NBUF, _CH, sp), jnp.float32),
            pltpu.VMEM((_NBUF, _CH, sp), jnp.float32),
            pltpu.SemaphoreType.DMA((_NBUF,)),
            pltpu.SemaphoreType.DMA((_NBUF,)),
        ],
    )(x2)
    return y2.reshape(b, c, h, w)


# speedup vs baseline: 3.8811x; 3.8811x over previous
import functools

import jax
import jax.numpy as jnp
from jax.experimental import pallas as pl
from jax.experimental.pallas import tpu as pltpu

_RATIO = 0.07651303708553314
_H_OFFSETS = (4, 2)
_W_COLS = (3, 1)

_CH = 16
_NBUF = 8


def _noise_points(h, w):
    mult = {}
    for ho, wv in zip(_H_OFFSETS, _W_COLS):
        rc = (h - 5 + ho, wv)
        mult[rc] = mult.get(rc, 0) + 1
    return tuple(sorted(mult.items()))


def _body(points, nchunks, x_hbm, o_hbm, ibuf, obuf, lsem, ssem):
    def load(i):
        return pltpu.make_async_copy(
            x_hbm.at[pl.ds(i * _CH, _CH)], ibuf.at[i % _NBUF],
            lsem.at[i % _NBUF])

    def store(i):
        return pltpu.make_async_copy(
            obuf.at[i % _NBUF], o_hbm.at[pl.ds(i * _CH, _CH)],
            ssem.at[i % _NBUF])

    for j in range(min(_NBUF, nchunks)):
        load(j).start()

    for i in range(nchunks):
        s = i % _NBUF
        load(i).wait()
        if i >= _NBUF:
            store(i - _NBUF).wait()
        v = ibuf[s]
        m = jnp.max(v, axis=2)
        m = jnp.max(m, axis=1, keepdims=True)
        m = m * jnp.float32(_RATIO)
        obuf[s] = v
        for (row, col), k in points:
            obuf[s, :, row:row + 1, col:col + 1] = (
                v[:, row:row + 1, col:col + 1]
                + jnp.float32(k) * m[:, :, None])
        store(i).start()
        if i + _NBUF < nchunks:
            load(i + _NBUF).start()

    for i in range(max(nchunks - _NBUF, 0), nchunks):
        store(i).wait()


@jax.jit
def kernel(x):
    b, c, h, w = x.shape
    n = b * c
    points = _noise_points(h, w)
    assert n % _CH == 0
    nchunks = n // _CH

    x3 = x.reshape(n, h, w)
    y3 = pl.pallas_call(
        functools.partial(_body, points, nchunks),
        in_specs=[pl.BlockSpec(memory_space=pl.ANY)],
        out_specs=pl.BlockSpec(memory_space=pl.ANY),
        out_shape=jax.ShapeDtypeStruct((n, h, w), x.dtype),
        scratch_shapes=[
            pltpu.VMEM((_NBUF, _CH, h, w), jnp.float32),
            pltpu.VMEM((_NBUF, _CH, h, w), jnp.float32),
            pltpu.SemaphoreType.DMA((_NBUF,)),
            pltpu.SemaphoreType.DMA((_NBUF,)),
        ],
    )(x3)
    return y3.reshape(b, c, h, w)


# speedup vs baseline: 3.8916x; 1.0027x over previous
import functools

import jax
import jax.numpy as jnp
from jax.experimental import pallas as pl
from jax.experimental.pallas import tpu as pltpu

_RATIO = 0.07651303708553314
_H_OFFSETS = (4, 2)
_W_COLS = (3, 1)

_CH = 32
_NBUF = 4


def _noise_points(h, w):
    mult = {}
    for ho, wv in zip(_H_OFFSETS, _W_COLS):
        rc = (h - 5 + ho, wv)
        mult[rc] = mult.get(rc, 0) + 1
    return tuple(sorted(mult.items()))


def _body(points, nchunks, x_hbm, o_hbm, ibuf, obuf, lsem, ssem):
    def load(i):
        return pltpu.make_async_copy(
            x_hbm.at[pl.ds(i * _CH, _CH)], ibuf.at[i % _NBUF],
            lsem.at[i % _NBUF])

    def store(i):
        return pltpu.make_async_copy(
            obuf.at[i % _NBUF], o_hbm.at[pl.ds(i * _CH, _CH)],
            ssem.at[i % _NBUF])

    for j in range(min(_NBUF, nchunks)):
        load(j).start()

    for i in range(nchunks):
        s = i % _NBUF
        load(i).wait()
        if i >= _NBUF:
            store(i - _NBUF).wait()
        v = ibuf[s]
        m = jnp.max(v, axis=2)
        m = jnp.max(m, axis=1, keepdims=True)
        m = m * jnp.float32(_RATIO)
        obuf[s] = v
        for (row, col), k in points:
            obuf[s, :, row:row + 1, col:col + 1] = (
                v[:, row:row + 1, col:col + 1]
                + jnp.float32(k) * m[:, :, None])
        store(i).start()
        if i + _NBUF < nchunks:
            load(i + _NBUF).start()

    for i in range(max(nchunks - _NBUF, 0), nchunks):
        store(i).wait()


@jax.jit
def kernel(x):
    b, c, h, w = x.shape
    n = b * c
    points = _noise_points(h, w)
    assert n % _CH == 0
    nchunks = n // _CH

    x3 = x.reshape(n, h, w)
    y3 = pl.pallas_call(
        functools.partial(_body, points, nchunks),
        in_specs=[pl.BlockSpec(memory_space=pl.ANY)],
        out_specs=pl.BlockSpec(memory_space=pl.ANY),
        out_shape=jax.ShapeDtypeStruct((n, h, w), x.dtype),
        scratch_shapes=[
            pltpu.VMEM((_NBUF, _CH, h, w), jnp.float32),
            pltpu.VMEM((_NBUF, _CH, h, w), jnp.float32),
            pltpu.SemaphoreType.DMA((_NBUF,)),
            pltpu.SemaphoreType.DMA((_NBUF,)),
        ],
    )(x3)
    return y3.reshape(b, c, h, w)
